# baseline (speedup 1.0000x reference)
import functools

import jax
import jax.numpy as jnp
from jax import lax
from jax.experimental import pallas as pl
from jax.experimental.pallas import tpu as pltpu
from jax.experimental.pallas import tpu_sc as plsc

VOCAB = 1000000
D_MODEL = 64
BATCH = 4096
HIST = 200

B = BATCH * HIST
IDX_W = 128
N_IDX_ROWS = B // IDX_W

_info = plsc.get_sparse_core_info()
NC, NS = _info.num_cores, _info.num_subcores
NW = NC * NS
ROWS_PER_W = N_IDX_ROWS // NW

K = 8
CHUNK = K * IDX_W
N_OUTER = ROWS_PER_W // K

mesh = plsc.VectorSubcoreMesh(core_axis_name="c", subcore_axis_name="s")


@functools.partial(
    pl.kernel,
    mesh=mesh,
    out_type=jax.ShapeDtypeStruct((B, D_MODEL), jnp.float32),
    scratch_types=[
        pltpu.VMEM((K, IDX_W), jnp.int32),
        pltpu.VMEM((CHUNK, D_MODEL), jnp.float32),
        pltpu.SemaphoreType.DMA,
    ],
    compiler_params=pltpu.CompilerParams(use_tc_tiling_on_sc=False),
)
def _gather_kernel(idx_hbm, table_hbm, out_hbm, idx_v, rows_v, sem):
    wid = lax.axis_index("s") * NC + lax.axis_index("c")
    row0 = wid * ROWS_PER_W

    def body(i, carry):
        irow = row0 + i * K
        pltpu.sync_copy(idx_hbm.at[pl.ds(irow, K)], idx_v)
        copies = []
        for j in range(K):
            copies.append(
                pltpu.async_copy(
                    table_hbm.at[idx_v.at[j]],
                    rows_v.at[pl.ds(j * IDX_W, IDX_W)],
                    sem,
                )
            )
        for c in copies:
            c.wait()
        pltpu.sync_copy(rows_v, out_hbm.at[pl.ds(irow * IDX_W, CHUNK)])
        return carry

    lax.fori_loop(0, N_OUTER, body, 0)


def kernel(x, table):
    idx = x.reshape(N_IDX_ROWS, IDX_W).astype(jnp.int32)
    out = _gather_kernel(idx, table)
    return out.reshape(BATCH, HIST, D_MODEL)


# speedup vs baseline: 1.0164x; 1.0164x over previous
import functools

import jax
import jax.numpy as jnp
from jax import lax
from jax.experimental import pallas as pl
from jax.experimental.pallas import tpu as pltpu
from jax.experimental.pallas import tpu_sc as plsc

VOCAB = 1000000
D_MODEL = 64
BATCH = 4096
HIST = 200

B = BATCH * HIST
IDX_W = 128
N_IDX_ROWS = B // IDX_W

_info = plsc.get_sparse_core_info()
NC, NS = _info.num_cores, _info.num_subcores
NW = NC * NS
ROWS_PER_W = N_IDX_ROWS // NW

K = 4
CHUNK = K * IDX_W
N_CHUNKS = ROWS_PER_W // K
NBUF = 2
N_OUTER = N_CHUNKS // NBUF

mesh = plsc.VectorSubcoreMesh(core_axis_name="c", subcore_axis_name="s")


@functools.partial(
    pl.kernel,
    mesh=mesh,
    out_type=jax.ShapeDtypeStruct((B, D_MODEL), jnp.float32),
    scratch_types=[
        pltpu.VMEM((NBUF, K, IDX_W), jnp.int32),
        pltpu.VMEM((NBUF, CHUNK, D_MODEL), jnp.float32),
        [pltpu.SemaphoreType.DMA] * NBUF,
        [pltpu.SemaphoreType.DMA] * NBUF,
        pltpu.SemaphoreType.DMA,
    ],
    compiler_params=pltpu.CompilerParams(use_tc_tiling_on_sc=False),
)
def _gather_kernel(idx_hbm, table_hbm, out_hbm, idx_v, rows_v,
                   sem_i, sem_o, sem_g):
    wid = lax.axis_index("s") * NC + lax.axis_index("c")
    row0 = wid * ROWS_PER_W

    for b in range(NBUF):
        pltpu.async_copy(
            idx_hbm.at[pl.ds(row0 + b * K, K)], idx_v.at[b], sem_i[b])

    def body(io, carry):
        for b in range(NBUF):
            i = io * NBUF + b
            irow = row0 + i * K
            @pl.when(io > 0)
            def _():
                pltpu.make_async_copy(
                    rows_v.at[b],
                    out_hbm.at[pl.ds(irow * IDX_W, CHUNK)],
                    sem_o[b],
                ).wait()
            pltpu.make_async_copy(
                idx_hbm.at[pl.ds(irow, K)], idx_v.at[b], sem_i[b]).wait()
            copies = []
            for j in range(K):
                copies.append(
                    pltpu.async_copy(
                        table_hbm.at[idx_v.at[b].at[j]],
                        rows_v.at[b].at[pl.ds(j * IDX_W, IDX_W)],
                        sem_g,
                    )
                )
            for c in copies:
                c.wait()
            pltpu.async_copy(
                rows_v.at[b],
                out_hbm.at[pl.ds(irow * IDX_W, CHUNK)],
                sem_o[b],
            )
            @pl.when(io < N_OUTER - 1)
            def _():
                pltpu.async_copy(
                    idx_hbm.at[pl.ds(irow + NBUF * K, K)],
                    idx_v.at[b], sem_i[b])
        return carry

    lax.fori_loop(0, N_OUTER, body, 0)

    for b in range(NBUF):
        i = (N_OUTER - 1) * NBUF + b
        pltpu.make_async_copy(
            rows_v.at[b],
            out_hbm.at[pl.ds((row0 + i * K) * IDX_W, CHUNK)],
            sem_o[b],
        ).wait()


def kernel(x, table):
    idx = x.reshape(N_IDX_ROWS, IDX_W).astype(jnp.int32)
    out = _gather_kernel(idx, table)
    return out.reshape(BATCH, HIST, D_MODEL)
